# baseline (speedup 1.0000x reference)
import functools

import jax
import jax.numpy as jnp
from jax import lax
from jax.experimental import pallas as pl
from jax.experimental.pallas import tpu as pltpu
from jax.experimental.pallas import tpu_sc as plsc

B = 16384
N_FEAT = 39
N_NUM = 13
N_CAT = 26
CARD = 100000
D = 16

NC = 2
NS = 16
NW = NC * NS
BPW = B // NW
GROUPS = BPW // 16
KCH = 128
NK = BPW // KCH
GPK = KCH // 16
CACHE = 64

MAGIC = 1.5 * 2.0**23


def _body(x_hbm, w_hbm, b_hbm, tab_hbm, out_hbm,
          xv, gidx, wv, bv, sharedv, cachev, cbuf, nbuf, mflag,
          gsem, cachesem, csem0, csem1, nsem0, nsem1):
    sid = lax.axis_index("s")
    wid = sid * NC + lax.axis_index("c")
    base = wid * BPW
    nsem = (nsem0, nsem1)
    csem = (csem0, csem1)

    pltpu.async_copy(
        tab_hbm.at[:, pl.ds(sid, 1), pl.ds(0, CACHE)],
        sharedv.at[:, pl.ds(sid, 1), :], cachesem)

    pltpu.sync_copy(x_hbm.at[:, pl.ds(base, BPW)], xv)
    pltpu.sync_copy(w_hbm, wv)
    pltpu.sync_copy(b_hbm, bv)

    def cat_idx_body(i, _):
        f = i >> 2
        k = i & (NK - 1)
        m = jnp.int32(0)
        for j in range(GPK):
            v = xv[N_NUM + f, pl.ds((k * GPK + j) * 16, 16)]
            v = jnp.where(v != v, 0.0, v)
            r = (v + MAGIC) - MAGIC
            r = jnp.minimum(r, float(CARD - 1))
            r = jnp.maximum(r, 0.0)
            ri = r.astype(jnp.int32)
            gidx[f, k, pl.ds(j * 16, 16)] = ri
            m = jnp.maximum(m, jnp.max(ri))
        mflag[f, k] = m
        return 0
    lax.fori_loop(0, N_CAT * NK, cat_idx_body, 0)

    for f in range(N_NUM):
        sel = f & 1
        wf = wv[f, :]
        bf = bv[f, :]
        if f >= 2:
            pltpu.make_async_copy(
                nbuf.at[sel], out_hbm.at[f - 2, :, pl.ds(base, BPW)],
                nsem[sel]).wait()

        def num_body(g, _):
            xg = xv[f, pl.ds(g * 16, 16)]
            for d in range(D):
                nbuf[sel, d, pl.ds(g * 16, 16)] = xg * wf[d] + bf[d]
            return 0
        lax.fori_loop(0, GROUPS, num_body, 0)
        pltpu.async_copy(
            nbuf.at[sel], out_hbm.at[f, :, pl.ds(base, BPW)], nsem[sel])

    pltpu.make_async_copy(
        tab_hbm.at[:, pl.ds(sid, 1), pl.ds(0, CACHE)],
        sharedv.at[:, pl.ds(sid, 1), :], cachesem).wait()
    plsc.subcore_barrier()
    pltpu.sync_copy(sharedv, cachev)

    def compute_cat(cf, buf):
        cf16 = jnp.full((16,), cf, jnp.int32)

        def grp_body(g, _):
            idxg = gidx[cf, g >> 3, pl.ds((g & 7) * 16, 16)]
            idxc = jnp.minimum(idxg, CACHE - 1)
            for d in range(D):
                val = plsc.load_gather(
                    cachev, [cf16, jnp.full((16,), d, jnp.int32), idxc])
                buf[d, pl.ds(g * 16, 16)] = val
            return 0
        lax.fori_loop(0, GROUPS, grp_body, 0)

        for k in range(NK):
            @pl.when(mflag[cf, k] >= CACHE)
            def _():
                for d in range(D):
                    pltpu.async_copy(
                        tab_hbm.at[cf, d].at[gidx.at[cf, k]],
                        buf.at[d, pl.ds(k * KCH, KCH)],
                        gsem)
                pltpu.make_async_copy(
                    tab_hbm.at[0, :, pl.ds(0, KCH)],
                    buf.at[:, pl.ds(0, KCH)], gsem).wait()

    def cat_pair(j, _):
        cf0 = 2 * j
        cf1 = 2 * j + 1

        @pl.when(j >= 1)
        def _():
            pltpu.make_async_copy(
                cbuf.at[0], out_hbm.at[N_NUM, :, pl.ds(base, BPW)],
                csem[0]).wait()
        compute_cat(cf0, cbuf.at[0])
        pltpu.async_copy(
            cbuf.at[0], out_hbm.at[N_NUM + cf0, :, pl.ds(base, BPW)], csem[0])

        @pl.when(j >= 1)
        def _():
            pltpu.make_async_copy(
                cbuf.at[1], out_hbm.at[N_NUM, :, pl.ds(base, BPW)],
                csem[1]).wait()
        compute_cat(cf1, cbuf.at[1])
        pltpu.async_copy(
            cbuf.at[1], out_hbm.at[N_NUM + cf1, :, pl.ds(base, BPW)], csem[1])
        return 0
    lax.fori_loop(0, N_CAT // 2, cat_pair, 0)

    for sel in range(2):
        pltpu.make_async_copy(
            nbuf.at[sel], out_hbm.at[0, :, pl.ds(base, BPW)],
            nsem[sel]).wait()
        pltpu.make_async_copy(
            cbuf.at[sel], out_hbm.at[N_NUM, :, pl.ds(base, BPW)],
            csem[sel]).wait()


@functools.partial(
    pl.kernel,
    out_type=jax.ShapeDtypeStruct((N_FEAT, D, B), jnp.float32),
    mesh=plsc.VectorSubcoreMesh(core_axis_name="c", subcore_axis_name="s"),
    compiler_params=pltpu.CompilerParams(
        use_tc_tiling_on_sc=False, needs_layout_passes=False),
    scratch_types=[
        pltpu.VMEM((N_FEAT, BPW), jnp.float32),
        pltpu.VMEM((N_CAT, NK, KCH), jnp.int32),
        pltpu.VMEM((N_NUM, D), jnp.float32),
        pltpu.VMEM((N_NUM, D), jnp.float32),
        pltpu.VMEM_SHARED((N_CAT, D, CACHE), jnp.float32),
        pltpu.VMEM((N_CAT, D, CACHE), jnp.float32),
        pltpu.VMEM((2, D, BPW), jnp.float32),
        pltpu.VMEM((2, D, BPW), jnp.float32),
        pltpu.SMEM((N_CAT, NK), jnp.int32),
        pltpu.SemaphoreType.DMA,
        pltpu.SemaphoreType.DMA,
        pltpu.SemaphoreType.DMA,
        pltpu.SemaphoreType.DMA,
        pltpu.SemaphoreType.DMA,
        pltpu.SemaphoreType.DMA,
    ],
)
def _sc_embed(x_hbm, w_hbm, b_hbm, tab_hbm, out_hbm,
              xv, gidx, wv, bv, sharedv, cachev, cbuf, nbuf, mflag,
              gsem, cachesem, csem0, csem1, nsem0, nsem1):
    _body(x_hbm, w_hbm, b_hbm, tab_hbm, out_hbm,
          xv, gidx, wv, bv, sharedv, cachev, cbuf, nbuf, mflag,
          gsem, cachesem, csem0, csem1, nsem0, nsem1)


def kernel(x, num_W, num_b, cat_tables):
    out_t = _sc_embed(x.T, num_W, num_b, cat_tables.transpose(0, 2, 1))
    return out_t.transpose(2, 0, 1)
